# baseline (speedup 1.0000x reference)
import functools

import jax
import jax.numpy as jnp
from jax import lax
from jax.experimental import pallas as pl
from jax.experimental.pallas import tpu as pltpu
from jax.experimental.pallas import tpu_sc as plsc

NC = 2
NS = 16
NW = NC * NS

PC = 80
AC = 40
RPT_G = 320
NP = 10240
RTILE = NP // NS
RCHUNK = 40


def _sc_mesh():
    return plsc.VectorSubcoreMesh(core_axis_name="c", subcore_axis_name="s")


def _make_sc_prep(n, d, e):
    cpt = e // (NW * PC)

    def body(x_hbm, dst_hbm, item_hbm, deg_out, h0_out,
             xidx_v, rows_v, dstbuf, hist_v, tmp_v, acc_v, hist_sh, sem):
        cc = lax.axis_index("c")
        s = lax.axis_index("s")
        w = cc * NS + s

        def zz(i, _):
            hist_v[pl.ds(i * 16, 16)] = jnp.zeros((16,), jnp.float32)
            return 0
        lax.fori_loop(0, NP // 16, zz, 0)

        base = jnp.minimum(w * RPT_G, n - RPT_G)
        pltpu.sync_copy(x_hbm.at[pl.ds(base, RPT_G)], xidx_v)
        for j in range(RPT_G // PC):
            pltpu.async_copy(
                item_hbm.at[xidx_v.at[pl.ds(j * PC, PC)]], rows_v, sem).wait()
            pltpu.sync_copy(rows_v, h0_out.at[pl.ds(base + j * PC, PC)])

        pltpu.sync_copy(dst_hbm.at[w], dstbuf)
        ones16 = jnp.ones((16,), jnp.float32)

        def dg(ci, _):
            for j in range(PC // 16):
                idx = dstbuf[ci, pl.ds(j * 16, 16)]
                plsc.addupdate_scatter(hist_v, [idx], ones16)
            return 0
        lax.fori_loop(0, cpt, dg, 0)

        pltpu.sync_copy(hist_v, hist_sh.at[pl.ds(s * NP, NP)])
        plsc.subcore_barrier()

        def za(i, _):
            acc_v[pl.ds(i * 16, 16)] = jnp.zeros((16,), jnp.float32)
            return 0
        lax.fori_loop(0, RTILE // 16, za, 0)
        for k in range(NS):
            pltpu.sync_copy(hist_sh.at[pl.ds(k * NP + s * RTILE, RTILE)], tmp_v)

            def aa(i, _):
                sl = pl.ds(i * 16, 16)
                acc_v[sl] = acc_v[sl] + tmp_v[sl]
                return 0
            lax.fori_loop(0, RTILE // 16, aa, 0)
        pltpu.sync_copy(acc_v, deg_out.at[pl.ds(cc * NP + s * RTILE, RTILE)])

    return pl.kernel(
        body,
        out_type=(
            jax.ShapeDtypeStruct((NC * NP,), jnp.float32),
            jax.ShapeDtypeStruct((n, d), jnp.float32),
        ),
        mesh=_sc_mesh(),
        compiler_params=pltpu.CompilerParams(needs_layout_passes=False),
        scratch_types=[
            pltpu.VMEM((RPT_G,), jnp.int32),
            pltpu.VMEM((PC, d), jnp.float32),
            pltpu.VMEM((cpt, PC), jnp.int32),
            pltpu.VMEM((NP,), jnp.float32),
            pltpu.VMEM((RTILE,), jnp.float32),
            pltpu.VMEM((RTILE,), jnp.float32),
            pltpu.VMEM_SHARED((NS * NP,), jnp.float32),
            pltpu.SemaphoreType.DMA,
        ],
    )


def _make_sc_agg(n, d, e, mode='real'):
    ept = e // NW
    SEC = 2000
    nsec = ept // SEC
    spc = SEC // AC

    def body(h_hbm, src_hbm, dst_hbm, out_hbm,
             srcbuf, dstbuf, rows, obuf, agg_sh, sg, ss):
        cc = lax.axis_index("c")
        s = lax.axis_index("s")
        w = cc * NS + s
        rb = s * RTILE

        def zz(t, _):
            i = t // (d // 16)
            j = t % (d // 16)
            obuf[i, pl.ds(j * 16, 16)] = jnp.zeros((16,), jnp.float32)
            return 0
        lax.fori_loop(0, RCHUNK * (d // 16), zz, 0)
        for k in range(RTILE // RCHUNK):
            pltpu.sync_copy(obuf, agg_sh.at[pl.ds(rb + k * RCHUNK, RCHUNK)])

        plsc.subcore_barrier()

        def drain(b, sems):
            pltpu.make_async_copy(h_hbm.at[pl.ds(0, AC)], rows[b], sems[b]).wait()

        def gath(buf_b, q):
            pltpu.async_copy(
                h_hbm.at[srcbuf.at[pl.ds(q * AC, AC)]], rows[buf_b], sg[buf_b])

        def scat(buf_b, q):
            pltpu.async_copy(
                rows[buf_b], agg_sh.at[dstbuf.at[pl.ds(q * AC, AC)]],
                ss[buf_b], add=True)

        for sec in range(nsec):
            off = w * ept + sec * SEC
            pltpu.sync_copy(src_hbm.at[pl.ds(off, SEC)], srcbuf)
            pltpu.sync_copy(dst_hbm.at[pl.ds(off, SEC)], dstbuf)
            if mode == 'gl':
                def ovs(i, _):
                    srcbuf[pl.ds(i * 16, 16)] = (
                        w * 312 + ((i * 16) % 296)
                        + lax.iota(jnp.int32, 16))
                    return 0
                lax.fori_loop(0, SEC // 16, ovs, 0)
            if mode == 'sl':
                def ovd(i, _):
                    dstbuf[pl.ds(i * 16, 16)] = (
                        rb + ((i * 16) % 624) + lax.iota(jnp.int32, 16))
                    return 0
                lax.fori_loop(0, SEC // 16, ovd, 0)

            gath(0, 0)
            gath(1, 1)
            drain(0, sg); scat(0, 0); gath(2, 2)
            drain(1, sg); scat(1, 1); gath(3, 3)

            def step(q, b):
                bn = (b + 2) % 4
                drain(b, sg)
                scat(b, q)
                drain(bn, ss)
                gath(bn, q + 2)

            def ed(t, _):
                q0 = 4 * t + 2
                for j in range(4):
                    step(q0 + j, (2 + j) % 4)
                return 0
            lax.fori_loop(0, (spc - 6) // 4, ed, 0)
            step(spc - 4, (spc - 4) % 4)
            step(spc - 3, (spc - 3) % 4)
            b = (spc - 2) % 4
            drain(b, sg); scat(b, spc - 2)
            b = (spc - 1) % 4
            drain(b, sg); scat(b, spc - 1)
            for b in range(4):
                drain(b, ss)

        plsc.subcore_barrier()

        for k in range(RTILE // RCHUNK):
            r0 = rb + k * RCHUNK
            pltpu.sync_copy(agg_sh.at[pl.ds(r0, RCHUNK)], obuf)
            pltpu.sync_copy(obuf, out_hbm.at[cc, pl.ds(r0, RCHUNK)])

    return pl.kernel(
        body,
        out_type=jax.ShapeDtypeStruct((NC, NP, d), jnp.float32),
        mesh=_sc_mesh(),
        scratch_types=[
            pltpu.VMEM((SEC,), jnp.int32),
            pltpu.VMEM((SEC,), jnp.int32),
            [pltpu.VMEM((AC, d), jnp.float32) for _ in range(4)],
            pltpu.VMEM((RCHUNK, d), jnp.float32),
            pltpu.VMEM_SHARED((NP, d), jnp.float32),
            [pltpu.SemaphoreType.DMA for _ in range(4)],
            [pltpu.SemaphoreType.DMA for _ in range(4)],
        ],
    )


def _tc_layer(h, parts, degp, Wl, Wr, b, W1=None, b1=None):
    n, d = h.shape
    R = 1000
    fused = W1 is not None

    def body(*refs):
        if fused:
            h_ref, p_ref, d_ref, wl, wr, bb, w1, b1r, o_ref = refs
        else:
            h_ref, p_ref, d_ref, wl, wr, bb, o_ref = refs
        deg = d_ref[0] + d_ref[1]
        inv = 1.0 / jnp.maximum(deg, 1.0)
        mean = (p_ref[0] + p_ref[1]) * inv
        acc = (jnp.dot(mean, wl[...], preferred_element_type=jnp.float32)
               + jnp.dot(h_ref[...], wr[...], preferred_element_type=jnp.float32)
               + bb[...])
        out = jnp.maximum(acc, 0.0)
        if fused:
            out = jnp.maximum(
                jnp.dot(out, w1[...], preferred_element_type=jnp.float32)
                + b1r[...], 0.0)
        o_ref[...] = out

    in_specs = [
        pl.BlockSpec((R, d), lambda i: (i, 0)),
        pl.BlockSpec((NC, R, d), lambda i: (0, i, 0)),
        pl.BlockSpec((NC, R, 1), lambda i: (0, i, 0)),
        pl.BlockSpec((d, d), lambda i: (0, 0)),
        pl.BlockSpec((d, d), lambda i: (0, 0)),
        pl.BlockSpec((1, d), lambda i: (0, 0)),
    ]
    args = [h, parts, degp, Wl, Wr, b.reshape(1, d)]
    if fused:
        in_specs += [pl.BlockSpec((d, d), lambda i: (0, 0)),
                     pl.BlockSpec((1, d), lambda i: (0, 0))]
        args += [W1, b1.reshape(1, d)]

    return pl.pallas_call(
        body,
        grid=(n // R,),
        in_specs=in_specs,
        out_specs=pl.BlockSpec((R, d), lambda i: (i, 0)),
        out_shape=jax.ShapeDtypeStruct((n, d), jnp.float32),
    )(*args)


def _tc_edge(et, W2, b2):
    e, d = et.shape
    RE = 4000

    def body(e_ref, w_ref, b_ref, o_ref):
        o_ref[...] = jnp.maximum(
            jnp.dot(e_ref[...], w_ref[...], preferred_element_type=jnp.float32)
            + b_ref[...], 0.0)

    return pl.pallas_call(
        body,
        grid=(e // RE,),
        in_specs=[
            pl.BlockSpec((RE, d), lambda i: (i, 0)),
            pl.BlockSpec((d, d), lambda i: (0, 0)),
            pl.BlockSpec((1, d), lambda i: (0, 0)),
        ],
        out_specs=pl.BlockSpec((RE, d), lambda i: (i, 0)),
        out_shape=jax.ShapeDtypeStruct((e, d), jnp.float32),
    )(et, W2, b2.reshape(1, d))


def kernel(x, edge_index, batch, item_table, edge_table,
           Wl1, Wr1, bc1, Wl2, Wr2, bc2, Wl3, Wr3, bc3,
           W1, b1, W2, b2):
    n, d = item_table.shape
    e = edge_table.shape[0]
    del batch

    x_flat = x.reshape(n).astype(jnp.int32)
    src_a = edge_index[0].astype(jnp.int32)
    dst_a = edge_index[1].astype(jnp.int32)
    dst_p = edge_index[1].reshape(NW, e // (NW * PC), PC).astype(jnp.int32)

    degp, h0 = _make_sc_prep(n, d, e)(x_flat, dst_p, item_table)
    h = h0
    for m in ('real', 'real', 'real', 'gl', 'gl', 'gl', 'sl', 'sl', 'sl'):
        p = _make_sc_agg(n, d, e, m)(h, src_a, dst_a)
        h = p[0, :n] * 0.03 + h * 0.5
    return (h, jnp.zeros((e, d), jnp.float32))


# speedup vs baseline: 2.4501x; 2.4501x over previous
import functools

import jax
import jax.numpy as jnp
from jax import lax
from jax.experimental import pallas as pl
from jax.experimental.pallas import tpu as pltpu
from jax.experimental.pallas import tpu_sc as plsc

NC = 2
NS = 16
NW = NC * NS

PC = 80
AC = 40
RPT_G = 320
NP = 10240
RTILE = NP // NS
RCHUNK = 40


def _sc_mesh():
    return plsc.VectorSubcoreMesh(core_axis_name="c", subcore_axis_name="s")


def _make_sc_prep(n, d, e):
    cpt = e // (NW * PC)

    def body(x_hbm, dst_hbm, item_hbm, deg_out, h0_out,
             xidx_v, rows_v, dstbuf, hist_v, tmp_v, acc_v, hist_sh, sem):
        cc = lax.axis_index("c")
        s = lax.axis_index("s")
        w = cc * NS + s

        def zz(i, _):
            hist_v[pl.ds(i * 16, 16)] = jnp.zeros((16,), jnp.float32)
            return 0
        lax.fori_loop(0, NP // 16, zz, 0)

        base = jnp.minimum(w * RPT_G, n - RPT_G)
        pltpu.sync_copy(x_hbm.at[pl.ds(base, RPT_G)], xidx_v)
        for j in range(RPT_G // PC):
            pltpu.async_copy(
                item_hbm.at[xidx_v.at[pl.ds(j * PC, PC)]], rows_v, sem).wait()
            pltpu.sync_copy(rows_v, h0_out.at[pl.ds(base + j * PC, PC)])

        pltpu.sync_copy(dst_hbm.at[w], dstbuf)
        ones16 = jnp.ones((16,), jnp.float32)

        def dg(ci, _):
            for j in range(PC // 16):
                idx = dstbuf[ci, pl.ds(j * 16, 16)]
                plsc.addupdate_scatter(hist_v, [idx], ones16)
            return 0
        lax.fori_loop(0, cpt, dg, 0)

        pltpu.sync_copy(hist_v, hist_sh.at[pl.ds(s * NP, NP)])
        plsc.subcore_barrier()

        def za(i, _):
            acc_v[pl.ds(i * 16, 16)] = jnp.zeros((16,), jnp.float32)
            return 0
        lax.fori_loop(0, RTILE // 16, za, 0)
        for k in range(NS):
            pltpu.sync_copy(hist_sh.at[pl.ds(k * NP + s * RTILE, RTILE)], tmp_v)

            def aa(i, _):
                sl = pl.ds(i * 16, 16)
                acc_v[sl] = acc_v[sl] + tmp_v[sl]
                return 0
            lax.fori_loop(0, RTILE // 16, aa, 0)
        pltpu.sync_copy(acc_v, deg_out.at[pl.ds(cc * NP + s * RTILE, RTILE)])

    return pl.kernel(
        body,
        out_type=(
            jax.ShapeDtypeStruct((NC * NP,), jnp.float32),
            jax.ShapeDtypeStruct((n, d), jnp.float32),
        ),
        mesh=_sc_mesh(),
        compiler_params=pltpu.CompilerParams(needs_layout_passes=False),
        scratch_types=[
            pltpu.VMEM((RPT_G,), jnp.int32),
            pltpu.VMEM((PC, d), jnp.float32),
            pltpu.VMEM((cpt, PC), jnp.int32),
            pltpu.VMEM((NP,), jnp.float32),
            pltpu.VMEM((RTILE,), jnp.float32),
            pltpu.VMEM((RTILE,), jnp.float32),
            pltpu.VMEM_SHARED((NS * NP,), jnp.float32),
            pltpu.SemaphoreType.DMA,
        ],
    )


def _make_sc_agg(n, d, e):
    ept = e // NW
    SEC = 2000
    nsec = ept // SEC
    spc = SEC // AC

    def body(h_hbm, src_hbm, dst_hbm, out_hbm,
             srcbuf, dstbuf, rows, obuf, agg_sh, sg, ss):
        cc = lax.axis_index("c")
        s = lax.axis_index("s")
        w = cc * NS + s
        rb = s * RTILE

        def zz(t, _):
            i = t // (d // 16)
            j = t % (d // 16)
            obuf[i, pl.ds(j * 16, 16)] = jnp.zeros((16,), jnp.float32)
            return 0
        lax.fori_loop(0, RCHUNK * (d // 16), zz, 0)
        for k in range(RTILE // RCHUNK):
            pltpu.sync_copy(obuf, agg_sh.at[pl.ds(rb + k * RCHUNK, RCHUNK)])

        plsc.subcore_barrier()

        def drain(b, sems):
            pltpu.make_async_copy(h_hbm.at[pl.ds(0, AC)], rows[b], sems[b]).wait()

        def gath(buf_b, q):
            pltpu.async_copy(
                h_hbm.at[srcbuf.at[pl.ds(q * AC, AC)]], rows[buf_b], sg[buf_b])

        def scat(buf_b, q):
            pltpu.async_copy(
                rows[buf_b], agg_sh.at[dstbuf.at[pl.ds(q * AC, AC)]],
                ss[buf_b], add=True)

        for sec in range(nsec):
            off = w * ept + sec * SEC
            pltpu.sync_copy(src_hbm.at[pl.ds(off, SEC)], srcbuf)
            pltpu.sync_copy(dst_hbm.at[pl.ds(off, SEC)], dstbuf)

            gath(0, 0)
            gath(1, 1)
            drain(0, sg); scat(0, 0); gath(2, 2)
            drain(1, sg); scat(1, 1); gath(3, 3)

            def step(q, b):
                bn = (b + 2) % 4
                drain(b, sg)
                scat(b, q)
                drain(bn, ss)
                gath(bn, q + 2)

            def ed(t, _):
                q0 = 4 * t + 2
                for j in range(4):
                    step(q0 + j, (2 + j) % 4)
                return 0
            lax.fori_loop(0, (spc - 6) // 4, ed, 0)
            step(spc - 4, (spc - 4) % 4)
            step(spc - 3, (spc - 3) % 4)
            b = (spc - 2) % 4
            drain(b, sg); scat(b, spc - 2)
            b = (spc - 1) % 4
            drain(b, sg); scat(b, spc - 1)
            for b in range(4):
                drain(b, ss)

        plsc.subcore_barrier()

        for k in range(RTILE // RCHUNK):
            r0 = rb + k * RCHUNK
            pltpu.sync_copy(agg_sh.at[pl.ds(r0, RCHUNK)], obuf)
            pltpu.sync_copy(obuf, out_hbm.at[cc, pl.ds(r0, RCHUNK)])

    return pl.kernel(
        body,
        out_type=jax.ShapeDtypeStruct((NC, NP, d), jnp.float32),
        mesh=_sc_mesh(),
        scratch_types=[
            pltpu.VMEM((SEC,), jnp.int32),
            pltpu.VMEM((SEC,), jnp.int32),
            [pltpu.VMEM((AC, d), jnp.float32) for _ in range(4)],
            pltpu.VMEM((RCHUNK, d), jnp.float32),
            pltpu.VMEM_SHARED((NP, d), jnp.float32),
            [pltpu.SemaphoreType.DMA for _ in range(4)],
            [pltpu.SemaphoreType.DMA for _ in range(4)],
        ],
    )


def _tc_layer(h, parts, degp, Wl, Wr, b, W1=None, b1=None):
    n, d = h.shape
    R = 1000
    fused = W1 is not None

    def body(*refs):
        if fused:
            h_ref, p_ref, d_ref, wl, wr, bb, w1, b1r, o_ref = refs
        else:
            h_ref, p_ref, d_ref, wl, wr, bb, o_ref = refs
        deg = d_ref[0] + d_ref[1]
        inv = 1.0 / jnp.maximum(deg, 1.0)
        mean = (p_ref[0] + p_ref[1]) * inv
        acc = (jnp.dot(mean, wl[...], preferred_element_type=jnp.float32)
               + jnp.dot(h_ref[...], wr[...], preferred_element_type=jnp.float32)
               + bb[...])
        out = jnp.maximum(acc, 0.0)
        if fused:
            out = jnp.maximum(
                jnp.dot(out, w1[...], preferred_element_type=jnp.float32)
                + b1r[...], 0.0)
        o_ref[...] = out

    in_specs = [
        pl.BlockSpec((R, d), lambda i: (i, 0)),
        pl.BlockSpec((NC, R, d), lambda i: (0, i, 0)),
        pl.BlockSpec((NC, R, 1), lambda i: (0, i, 0)),
        pl.BlockSpec((d, d), lambda i: (0, 0)),
        pl.BlockSpec((d, d), lambda i: (0, 0)),
        pl.BlockSpec((1, d), lambda i: (0, 0)),
    ]
    args = [h, parts, degp, Wl, Wr, b.reshape(1, d)]
    if fused:
        in_specs += [pl.BlockSpec((d, d), lambda i: (0, 0)),
                     pl.BlockSpec((1, d), lambda i: (0, 0))]
        args += [W1, b1.reshape(1, d)]

    return pl.pallas_call(
        body,
        grid=(n // R,),
        in_specs=in_specs,
        out_specs=pl.BlockSpec((R, d), lambda i: (i, 0)),
        out_shape=jax.ShapeDtypeStruct((n, d), jnp.float32),
    )(*args)


def _tc_edge(et, W2, b2):
    e, d = et.shape
    RE = 4000

    def body(e_ref, w_ref, b_ref, o_ref):
        o_ref[...] = jnp.maximum(
            jnp.dot(e_ref[...], w_ref[...], preferred_element_type=jnp.float32)
            + b_ref[...], 0.0)

    return pl.pallas_call(
        body,
        grid=(e // RE,),
        in_specs=[
            pl.BlockSpec((RE, d), lambda i: (i, 0)),
            pl.BlockSpec((d, d), lambda i: (0, 0)),
            pl.BlockSpec((1, d), lambda i: (0, 0)),
        ],
        out_specs=pl.BlockSpec((RE, d), lambda i: (i, 0)),
        out_shape=jax.ShapeDtypeStruct((e, d), jnp.float32),
    )(et, W2, b2.reshape(1, d))


def kernel(x, edge_index, batch, item_table, edge_table,
           Wl1, Wr1, bc1, Wl2, Wr2, bc2, Wl3, Wr3, bc3,
           W1, b1, W2, b2):
    n, d = item_table.shape
    e = edge_table.shape[0]
    del batch

    x_flat = x.reshape(n).astype(jnp.int32)
    src_a = edge_index[0].astype(jnp.int32)
    dst_a = edge_index[1].astype(jnp.int32)
    dst_p = edge_index[1].reshape(NW, e // (NW * PC), PC).astype(jnp.int32)

    degp, h0 = _make_sc_prep(n, d, e)(x_flat, dst_p, item_table)
    degp = degp.reshape(NC, NP, 1)
    agg = _make_sc_agg(n, d, e)

    p1 = agg(h0, src_a, dst_a)
    e_out = _tc_edge(edge_table, W2, b2)
    h1 = _tc_layer(h0, p1, degp, Wl1, Wr1, bc1)
    p2 = agg(h1, src_a, dst_a)
    h2 = _tc_layer(h1, p2, degp, Wl2, Wr2, bc2)
    p3 = agg(h2, src_a, dst_a)
    h3 = _tc_layer(h2, p3, degp, Wl3, Wr3, bc3, W1, b1)
    return (h3, e_out)


# speedup vs baseline: 2.4716x; 1.0088x over previous
import functools

import jax
import jax.numpy as jnp
from jax import lax
from jax.experimental import pallas as pl
from jax.experimental.pallas import tpu as pltpu
from jax.experimental.pallas import tpu_sc as plsc

NC = 2
NS = 16
NW = NC * NS

PC = 80
AC = 40
RPT_G = 320
NP = 10240
RTILE = NP // NS
RCHUNK = 40


def _sc_mesh():
    return plsc.VectorSubcoreMesh(core_axis_name="c", subcore_axis_name="s")


def _make_sc_prep(n, d, e):
    cpt = e // (NW * PC)

    def body(x_hbm, dst_hbm, item_hbm, deg_out, h0_out,
             xidx_v, rows_v, dstbuf, hist_v, tmp_v, acc_v, hist_sh, sem):
        cc = lax.axis_index("c")
        s = lax.axis_index("s")
        w = cc * NS + s

        def zz(i, _):
            hist_v[pl.ds(i * 16, 16)] = jnp.zeros((16,), jnp.float32)
            return 0
        lax.fori_loop(0, NP // 16, zz, 0)

        base = jnp.minimum(w * RPT_G, n - RPT_G)
        pltpu.sync_copy(x_hbm.at[pl.ds(base, RPT_G)], xidx_v)
        for j in range(RPT_G // PC):
            pltpu.async_copy(
                item_hbm.at[xidx_v.at[pl.ds(j * PC, PC)]], rows_v, sem).wait()
            pltpu.sync_copy(rows_v, h0_out.at[pl.ds(base + j * PC, PC)])

        pltpu.sync_copy(dst_hbm.at[w], dstbuf)
        ones16 = jnp.ones((16,), jnp.float32)

        def dg(ci, _):
            for j in range(PC // 16):
                idx = dstbuf[ci, pl.ds(j * 16, 16)]
                plsc.addupdate_scatter(hist_v, [idx], ones16)
            return 0
        lax.fori_loop(0, cpt, dg, 0)

        pltpu.sync_copy(hist_v, hist_sh.at[pl.ds(s * NP, NP)])
        plsc.subcore_barrier()

        def za(i, _):
            acc_v[pl.ds(i * 16, 16)] = jnp.zeros((16,), jnp.float32)
            return 0
        lax.fori_loop(0, RTILE // 16, za, 0)
        for k in range(NS):
            pltpu.sync_copy(hist_sh.at[pl.ds(k * NP + s * RTILE, RTILE)], tmp_v)

            def aa(i, _):
                sl = pl.ds(i * 16, 16)
                acc_v[sl] = acc_v[sl] + tmp_v[sl]
                return 0
            lax.fori_loop(0, RTILE // 16, aa, 0)
        pltpu.sync_copy(acc_v, deg_out.at[pl.ds(cc * NP + s * RTILE, RTILE)])

    return pl.kernel(
        body,
        out_type=(
            jax.ShapeDtypeStruct((NC * NP,), jnp.float32),
            jax.ShapeDtypeStruct((n, d), jnp.float32),
        ),
        mesh=_sc_mesh(),
        compiler_params=pltpu.CompilerParams(needs_layout_passes=False),
        scratch_types=[
            pltpu.VMEM((RPT_G,), jnp.int32),
            pltpu.VMEM((PC, d), jnp.float32),
            pltpu.VMEM((cpt, PC), jnp.int32),
            pltpu.VMEM((NP,), jnp.float32),
            pltpu.VMEM((RTILE,), jnp.float32),
            pltpu.VMEM((RTILE,), jnp.float32),
            pltpu.VMEM_SHARED((NS * NP,), jnp.float32),
            pltpu.SemaphoreType.DMA,
        ],
    )


def _make_sc_agg(n, d, e):
    ept = e // NW
    SEC = 2000
    nsec = ept // SEC
    spc = SEC // AC

    def body(h_hbm, src_hbm, dst_hbm, out_hbm,
             srcbuf, dstbuf, rows, obuf, agg_sh, sg, ss):
        cc = lax.axis_index("c")
        s = lax.axis_index("s")
        w = cc * NS + s
        rb = s * RTILE

        def zz(t, _):
            i = t // (d // 16)
            j = t % (d // 16)
            obuf[i, pl.ds(j * 16, 16)] = jnp.zeros((16,), jnp.float32)
            return 0
        lax.fori_loop(0, RCHUNK * (d // 16), zz, 0)
        for k in range(RTILE // RCHUNK):
            pltpu.sync_copy(obuf, agg_sh.at[pl.ds(rb + k * RCHUNK, RCHUNK)])

        plsc.subcore_barrier()

        def drain(b, sems):
            pltpu.make_async_copy(h_hbm.at[pl.ds(0, AC)], rows[b], sems[b]).wait()

        def gath(buf_b, q):
            pltpu.async_copy(
                h_hbm.at[srcbuf.at[pl.ds(q * AC, AC)]], rows[buf_b], sg[buf_b])

        def scat(buf_b, q):
            pltpu.async_copy(
                rows[buf_b], agg_sh.at[dstbuf.at[pl.ds(q * AC, AC)]],
                ss[buf_b], add=True)

        for sec in range(nsec):
            off = w * ept + sec * SEC
            pltpu.sync_copy(src_hbm.at[pl.ds(off, SEC)], srcbuf)
            pltpu.sync_copy(dst_hbm.at[pl.ds(off, SEC)], dstbuf)

            gath(0, 0)
            gath(1, 1)
            drain(0, sg); scat(0, 0); gath(2, 2)
            drain(1, sg); scat(1, 1); gath(3, 3)

            def step(q, b):
                bn = (b + 2) % 4
                drain(b, sg)
                scat(b, q)
                drain(bn, ss)
                gath(bn, q + 2)

            def ed(t, _):
                q0 = 4 * t + 2
                for j in range(4):
                    step(q0 + j, (2 + j) % 4)
                return 0
            lax.fori_loop(0, (spc - 6) // 4, ed, 0)
            step(spc - 4, (spc - 4) % 4)
            step(spc - 3, (spc - 3) % 4)
            b = (spc - 2) % 4
            drain(b, sg); scat(b, spc - 2)
            b = (spc - 1) % 4
            drain(b, sg); scat(b, spc - 1)
            for b in range(4):
                drain(b, ss)

        plsc.subcore_barrier()

        pltpu.sync_copy(agg_sh.at[pl.ds(rb, RTILE)],
                        out_hbm.at[cc, pl.ds(rb, RTILE)])

    return pl.kernel(
        body,
        out_type=jax.ShapeDtypeStruct((NC, NP, d), jnp.float32),
        mesh=_sc_mesh(),
        scratch_types=[
            pltpu.VMEM((SEC,), jnp.int32),
            pltpu.VMEM((SEC,), jnp.int32),
            [pltpu.VMEM((AC, d), jnp.float32) for _ in range(4)],
            pltpu.VMEM((RCHUNK, d), jnp.float32),
            pltpu.VMEM_SHARED((NP, d), jnp.float32),
            [pltpu.SemaphoreType.DMA for _ in range(4)],
            [pltpu.SemaphoreType.DMA for _ in range(4)],
        ],
    )


def _tc_layer(h, parts, degp, Wl, Wr, b, W1=None, b1=None):
    n, d = h.shape
    R = 1000
    fused = W1 is not None

    def body(*refs):
        if fused:
            h_ref, p_ref, d_ref, wl, wr, bb, w1, b1r, o_ref = refs
        else:
            h_ref, p_ref, d_ref, wl, wr, bb, o_ref = refs
        deg = d_ref[0] + d_ref[1]
        inv = 1.0 / jnp.maximum(deg, 1.0)
        mean = (p_ref[0] + p_ref[1]) * inv
        acc = (jnp.dot(mean, wl[...], preferred_element_type=jnp.float32)
               + jnp.dot(h_ref[...], wr[...], preferred_element_type=jnp.float32)
               + bb[...])
        out = jnp.maximum(acc, 0.0)
        if fused:
            out = jnp.maximum(
                jnp.dot(out, w1[...], preferred_element_type=jnp.float32)
                + b1r[...], 0.0)
        o_ref[...] = out

    in_specs = [
        pl.BlockSpec((R, d), lambda i: (i, 0)),
        pl.BlockSpec((NC, R, d), lambda i: (0, i, 0)),
        pl.BlockSpec((NC, R, 1), lambda i: (0, i, 0)),
        pl.BlockSpec((d, d), lambda i: (0, 0)),
        pl.BlockSpec((d, d), lambda i: (0, 0)),
        pl.BlockSpec((1, d), lambda i: (0, 0)),
    ]
    args = [h, parts, degp, Wl, Wr, b.reshape(1, d)]
    if fused:
        in_specs += [pl.BlockSpec((d, d), lambda i: (0, 0)),
                     pl.BlockSpec((1, d), lambda i: (0, 0))]
        args += [W1, b1.reshape(1, d)]

    return pl.pallas_call(
        body,
        grid=(n // R,),
        in_specs=in_specs,
        out_specs=pl.BlockSpec((R, d), lambda i: (i, 0)),
        out_shape=jax.ShapeDtypeStruct((n, d), jnp.float32),
    )(*args)


def _tc_edge(et, W2, b2):
    e, d = et.shape
    RE = 4000

    def body(e_ref, w_ref, b_ref, o_ref):
        o_ref[...] = jnp.maximum(
            jnp.dot(e_ref[...], w_ref[...], preferred_element_type=jnp.float32)
            + b_ref[...], 0.0)

    return pl.pallas_call(
        body,
        grid=(e // RE,),
        in_specs=[
            pl.BlockSpec((RE, d), lambda i: (i, 0)),
            pl.BlockSpec((d, d), lambda i: (0, 0)),
            pl.BlockSpec((1, d), lambda i: (0, 0)),
        ],
        out_specs=pl.BlockSpec((RE, d), lambda i: (i, 0)),
        out_shape=jax.ShapeDtypeStruct((e, d), jnp.float32),
    )(et, W2, b2.reshape(1, d))


def kernel(x, edge_index, batch, item_table, edge_table,
           Wl1, Wr1, bc1, Wl2, Wr2, bc2, Wl3, Wr3, bc3,
           W1, b1, W2, b2):
    n, d = item_table.shape
    e = edge_table.shape[0]
    del batch

    x_flat = x.reshape(n).astype(jnp.int32)
    src_a = edge_index[0].astype(jnp.int32)
    dst_a = edge_index[1].astype(jnp.int32)
    dst_p = edge_index[1].reshape(NW, e // (NW * PC), PC).astype(jnp.int32)

    degp, h0 = _make_sc_prep(n, d, e)(x_flat, dst_p, item_table)
    degp = degp.reshape(NC, NP, 1)
    agg = _make_sc_agg(n, d, e)

    p1 = agg(h0, src_a, dst_a)
    e_out = _tc_edge(edge_table, W2, b2)
    h1 = _tc_layer(h0, p1, degp, Wl1, Wr1, bc1)
    p2 = agg(h1, src_a, dst_a)
    h2 = _tc_layer(h1, p2, degp, Wl2, Wr2, bc2)
    p3 = agg(h2, src_a, dst_a)
    h3 = _tc_layer(h2, p3, degp, Wl3, Wr3, bc3, W1, b1)
    return (h3, e_out)
